# baseline (speedup 1.0000x reference)
import jax
import jax.numpy as jnp
from jax.experimental import pallas as pl
from jax.experimental.pallas import tpu as pltpu

_CH = 256
_NBUF = 4


def _make_dma_kernel(B, S, D):
    nch = B * S // _CH
    per_b = S // _CH

    def _kern(tid_ref, x_hbm, p_hbm, o_hbm, bufs, rowbuf, sems, rowsem):
        ins = []
        outs = []
        for i in range(nch):
            b, r0 = i // per_b, (i % per_b) * _CH
            slot = i % _NBUF
            src = x_hbm.at[pl.ds(b, 1), pl.ds(r0, _CH), :]
            dst = o_hbm.at[pl.ds(b, 1), pl.ds(r0, _CH), :]
            buf = bufs.at[slot]
            ins.append(pltpu.make_async_copy(src, buf, sems.at[0, slot]))
            outs.append(pltpu.make_async_copy(buf, dst, sems.at[1, slot]))

        rowin = []
        rowout = []
        for b in range(B):
            rowin.append(
                pltpu.make_async_copy(
                    p_hbm.at[pl.ds(tid_ref[b], 1), :],
                    rowbuf.at[pl.ds(b, 1), :],
                    rowsem,
                )
            )
            rowout.append(
                pltpu.make_async_copy(
                    rowbuf.at[pl.ds(b, 1), :],
                    o_hbm.at[b, pl.ds(S, 1), :],
                    rowsem,
                )
            )
        for c in rowin:
            c.start()

        for j in range(_NBUF):
            ins[j].start()
        for i in range(nch):
            if i >= _NBUF:
                outs[i - _NBUF].wait()
                ins[i].start()
            ins[i].wait()
            outs[i].start()
        for c in rowin:
            c.wait()
        for c in rowout:
            c.start()
        for j in range(nch - _NBUF, nch):
            outs[j].wait()
        for c in rowout:
            c.wait()

    return _kern


def kernel(x, task_id, prompt):
    B, S, D = x.shape
    task_id32 = task_id.astype(jnp.int32)

    out = pl.pallas_call(
        _make_dma_kernel(B, S, D),
        in_specs=[
            pl.BlockSpec(memory_space=pltpu.MemorySpace.SMEM),
            pl.BlockSpec(memory_space=pltpu.MemorySpace.HBM),
            pl.BlockSpec(memory_space=pltpu.MemorySpace.HBM),
        ],
        out_specs=pl.BlockSpec(memory_space=pltpu.MemorySpace.HBM),
        out_shape=jax.ShapeDtypeStruct((B, S + 1, D), x.dtype),
        scratch_shapes=[
            pltpu.VMEM((_NBUF, 1, _CH, D), x.dtype),
            pltpu.VMEM((B, D), x.dtype),
            pltpu.SemaphoreType.DMA((2, _NBUF)),
            pltpu.SemaphoreType.DMA,
        ],
    )(task_id32, x, prompt)
    return (out, task_id)


# speedup vs baseline: 1.1192x; 1.1192x over previous
import functools

import jax
import jax.numpy as jnp
from jax import lax
from jax.experimental import pallas as pl
from jax.experimental.pallas import tpu as pltpu
from jax.experimental.pallas import tpu_sc as plsc

_B, _D = 4, 1024


@functools.partial(
    pl.kernel,
    mesh=plsc.VectorSubcoreMesh(core_axis_name="c", subcore_axis_name="s"),
    out_type=jax.ShapeDtypeStruct((_B, 1, _D), jnp.float32),
    scratch_types=[
        pltpu.VMEM((_B,), jnp.int32),
        pltpu.VMEM((_B, _D), jnp.float32),
        pltpu.SemaphoreType.DMA,
    ],
)
def _sc_gather(tid_hbm, p_hbm, rows_hbm, idx_v, rows_v, sem):
    c = lax.axis_index("c")
    s = lax.axis_index("s")
    wid = s * 2 + c

    @pl.when(wid == 0)
    def _():
        pltpu.sync_copy(tid_hbm, idx_v)
        pltpu.async_copy(p_hbm.at[idx_v], rows_v, sem).wait()
        pltpu.sync_copy(rows_v, rows_hbm.at[:, 0, :])


def _concat_kernel(x_ref, p_ref, o_ref):
    seq = x_ref.shape[1]
    o_ref[0, :seq, :] = x_ref[0]
    o_ref[0, seq, :] = p_ref[0, 0]


def kernel(x, task_id, prompt):
    B, S, D = x.shape
    task_id32 = task_id.astype(jnp.int32)

    rows = _sc_gather(task_id32, prompt)

    out = pl.pallas_call(
        _concat_kernel,
        grid=(B,),
        in_specs=[
            pl.BlockSpec((1, S, D), lambda b: (b, 0, 0)),
            pl.BlockSpec((1, 1, D), lambda b: (b, 0, 0)),
        ],
        out_specs=pl.BlockSpec((1, S + 1, D), lambda b: (b, 0, 0)),
        out_shape=jax.ShapeDtypeStruct((B, S + 1, D), x.dtype),
    )(x, rows)
    return (out, task_id)


# speedup vs baseline: 1.1369x; 1.0158x over previous
import functools

import jax
import jax.numpy as jnp
from jax.experimental import pallas as pl
from jax.experimental.pallas import tpu as pltpu

_BS = 256


def _concat_kernel(tid_ref, x_ref, p_ref, o_ref, *, ns, bs):
    s = pl.program_id(1)
    o_ref[0, pl.ds(s * bs, bs), :] = x_ref[0]

    @pl.when(s == ns - 1)
    def _row():
        o_ref[0, ns * bs, :] = p_ref[0, 0]


def kernel(x, task_id, prompt):
    B, S, D = x.shape
    ns = S // _BS
    task_id32 = task_id.astype(jnp.int32)
    prompt3 = prompt.reshape(prompt.shape[0], 1, prompt.shape[1])

    grid_spec = pltpu.PrefetchScalarGridSpec(
        num_scalar_prefetch=1,
        grid=(B, ns),
        in_specs=[
            pl.BlockSpec((1, _BS, D), lambda b, s, tid: (b, s, 0)),
            pl.BlockSpec((1, 1, D), lambda b, s, tid: (tid[b], 0, 0)),
        ],
        out_specs=pl.BlockSpec((1, S + 1, D), lambda b, s, tid: (b, 0, 0)),
    )

    out = pl.pallas_call(
        functools.partial(_concat_kernel, ns=ns, bs=_BS),
        grid_spec=grid_spec,
        out_shape=jax.ShapeDtypeStruct((B, S + 1, D), x.dtype),
    )(task_id32, x, prompt3)
    return (out, task_id)


# speedup vs baseline: 1.2919x; 1.1363x over previous
import functools

import jax
import jax.numpy as jnp
from jax.experimental import pallas as pl
from jax.experimental.pallas import tpu as pltpu

_DS = 512


def _concat_kernel(tid_ref, x_ref, p_ref, o_ref):
    seq = x_ref.shape[1]
    o_ref[0, :seq, :] = x_ref[0]
    o_ref[0, seq, :] = p_ref[0, 0]


def kernel(x, task_id, prompt):
    B, S, D = x.shape
    nd = D // _DS
    task_id32 = task_id.astype(jnp.int32)
    prompt3 = prompt.reshape(prompt.shape[0], 1, prompt.shape[1])

    grid_spec = pltpu.PrefetchScalarGridSpec(
        num_scalar_prefetch=1,
        grid=(B, nd),
        in_specs=[
            pl.BlockSpec((1, S, _DS), lambda b, d, tid: (b, 0, d)),
            pl.BlockSpec((1, 1, _DS), lambda b, d, tid: (tid[b], 0, d)),
        ],
        out_specs=pl.BlockSpec((1, S + 1, _DS), lambda b, d, tid: (b, 0, d)),
    )

    out = pl.pallas_call(
        _concat_kernel,
        grid_spec=grid_spec,
        out_shape=jax.ShapeDtypeStruct((B, S + 1, D), x.dtype),
    )(task_id32, x, prompt3)
    return (out, task_id)


# speedup vs baseline: 1.3225x; 1.0237x over previous
import jax
import jax.numpy as jnp
from jax.experimental import pallas as pl
from jax.experimental.pallas import tpu as pltpu


def _concat_kernel(task_id_ref, x_ref, p_ref, o_ref):
    seq = x_ref.shape[1]
    o_ref[0, :seq, :] = x_ref[0]
    o_ref[0, seq, :] = p_ref[0, 0]


def kernel(x, task_id, prompt):
    B, S, D = x.shape
    task_id32 = task_id.astype(jnp.int32)
    prompt3 = prompt.reshape(prompt.shape[0], 1, prompt.shape[1])

    grid_spec = pltpu.PrefetchScalarGridSpec(
        num_scalar_prefetch=1,
        grid=(B,),
        in_specs=[
            pl.BlockSpec((1, S, D), lambda b, tid: (b, 0, 0)),
            pl.BlockSpec((1, 1, D), lambda b, tid: (tid[b], 0, 0)),
        ],
        out_specs=pl.BlockSpec((1, S + 1, D), lambda b, tid: (b, 0, 0)),
    )

    out = pl.pallas_call(
        _concat_kernel,
        grid_spec=grid_spec,
        out_shape=jax.ShapeDtypeStruct((B, S + 1, D), x.dtype),
        compiler_params=pltpu.CompilerParams(skip_device_barrier=True),
    )(task_id32, x, prompt3)
    return (out, task_id)
